# baseline (speedup 1.0000x reference)
import functools

import jax
import jax.numpy as jnp
from jax import lax
from jax.experimental import pallas as pl
from jax.experimental.pallas import tpu as pltpu
from jax.experimental.pallas import tpu_sc as plsc

B, S, D = 4, 8192, 1024
NC, NS = 2, 16
NW = NC * NS
ROWS_PER_W = S // NW
R = 32
NCHUNK = ROWS_PER_W // R
LANES = 16
GU = 8

_mesh = plsc.VectorSubcoreMesh(core_axis_name="c", subcore_axis_name="s")


@functools.partial(
    pl.kernel,
    mesh=_mesh,
    out_type=jax.ShapeDtypeStruct((B, S, D), jnp.float32),
    scratch_types=[
        pltpu.VMEM((R, D), jnp.float32),
        pltpu.VMEM((R, D), jnp.float32),
    ],
)
def _pe_add(x_hbm, pe_hbm, out_hbm, x_buf, pe_buf):
    wid = lax.axis_index("s") * NC + lax.axis_index("c")
    base = wid * ROWS_PER_W

    def chunk_body(c, carry):
        row0 = base + c * R
        pltpu.sync_copy(pe_hbm.at[pl.ds(row0, R)], pe_buf)
        for b in range(B):
            pltpu.sync_copy(x_hbm.at[b, pl.ds(row0, R)], x_buf)

            def row_body(i, rc):
                def grp_body(j, gc):
                    for u in range(GU):
                        sl = pl.ds(j * (LANES * GU) + u * LANES, LANES)
                        x_buf[i, sl] = x_buf[i, sl] + pe_buf[i, sl]
                    return gc

                return lax.fori_loop(0, D // (LANES * GU), grp_body, rc)

            lax.fori_loop(0, R, row_body, 0)
            pltpu.sync_copy(x_buf, out_hbm.at[b, pl.ds(row0, R)])
        return carry

    lax.fori_loop(0, NCHUNK, chunk_body, 0)


def kernel(x, pe_table):
    return _pe_add(x, pe_table)


# speedup vs baseline: 1.4307x; 1.4307x over previous
import functools

import jax
import jax.numpy as jnp
from jax import lax
from jax.experimental import pallas as pl
from jax.experimental.pallas import tpu as pltpu
from jax.experimental.pallas import tpu_sc as plsc

B, S, D = 4, 8192, 1024
NC, NS = 2, 16
NW = NC * NS
ROWS_PER_W = S // NW
R = 16
NCHUNK = ROWS_PER_W // R
CHUNK = R * D
LANES = 16
GU = 8

_mesh = plsc.VectorSubcoreMesh(core_axis_name="c", subcore_axis_name="s")


@functools.partial(
    pl.kernel,
    mesh=_mesh,
    out_type=jax.ShapeDtypeStruct((B, S * D), jnp.float32),
    scratch_types=[
        pltpu.VMEM((CHUNK,), jnp.float32),
        pltpu.VMEM((CHUNK,), jnp.float32),
        pltpu.VMEM((CHUNK,), jnp.float32),
        pltpu.VMEM((CHUNK,), jnp.float32),
        pltpu.VMEM((CHUNK,), jnp.float32),
        pltpu.VMEM((CHUNK,), jnp.float32),
        pltpu.SemaphoreType.DMA,
        pltpu.SemaphoreType.DMA,
        pltpu.SemaphoreType.DMA,
        pltpu.SemaphoreType.DMA,
        pltpu.SemaphoreType.DMA,
        pltpu.SemaphoreType.DMA,
    ],
)
def _pe_add(x_hbm, pe_hbm, out_hbm, xb0, xb1, ob0, ob1, pb0, pb1,
            sx0, sx1, so0, so1, sp0, sp1):
    wid = lax.axis_index("s") * NC + lax.axis_index("c")
    ebase = wid * (ROWS_PER_W * D)

    xb, ob, pb = [xb0, xb1], [ob0, ob1], [pb0, pb1]
    sx, so, sp = [sx0, sx1], [so0, so1], [sp0, sp1]

    steps = [(c, b) for c in range(NCHUNK) for b in range(B)]
    T = len(steps)

    def xload(t):
        c, b = steps[t]
        return pltpu.make_async_copy(
            x_hbm.at[b, pl.ds(ebase + c * CHUNK, CHUNK)], xb[t % 2], sx[t % 2])

    def peload(c):
        return pltpu.make_async_copy(
            pe_hbm.at[pl.ds(ebase + c * CHUNK, CHUNK)], pb[c % 2], sp[c % 2])

    def store(t):
        c, b = steps[t]
        return pltpu.make_async_copy(
            ob[t % 2], out_hbm.at[b, pl.ds(ebase + c * CHUNK, CHUNK)], so[t % 2])

    pending_x = {}
    pending_pe = {}
    pending_o = {}

    pending_pe[0] = peload(0)
    pending_pe[0].start()
    pending_x[0] = xload(0)
    pending_x[0].start()

    for t in range(T):
        c, b = steps[t]
        pending_x.pop(t).wait()
        if b == 0:
            pending_pe.pop(c).wait()
        if t + 1 < T:
            pending_x[t + 1] = xload(t + 1)
            pending_x[t + 1].start()
        if b == 0 and c + 1 < NCHUNK:
            pending_pe[c + 1] = peload(c + 1)
            pending_pe[c + 1].start()
        if t >= 2:
            pending_o.pop(t - 2).wait()

        xbuf, obuf, pbuf = xb[t % 2], ob[t % 2], pb[c % 2]

        def grp_body(g, carry, xbuf=xbuf, obuf=obuf, pbuf=pbuf):
            base = g * (LANES * GU)
            for u in range(GU):
                sl = pl.ds(base + u * LANES, LANES)
                obuf[sl] = xbuf[sl] + pbuf[sl]
            return carry

        lax.fori_loop(0, CHUNK // (LANES * GU), grp_body, 0)

        pending_o[t] = store(t)
        pending_o[t].start()

    pending_o.pop(T - 2).wait()
    pending_o.pop(T - 1).wait()


def kernel(x, pe_table):
    out = _pe_add(x.reshape(B, S * D), pe_table.reshape(S * D))
    return out.reshape(B, S, D)


# speedup vs baseline: 3.5535x; 2.4838x over previous
import functools

import jax
import jax.numpy as jnp
from jax import lax
from jax.experimental import pallas as pl
from jax.experimental.pallas import tpu as pltpu
from jax.experimental.pallas import tpu_sc as plsc

B, S, D = 4, 8192, 1024
NC, NS = 2, 16
NW = NC * NS
ROWS_PER_W = S // NW
R = 16
NCHUNK = ROWS_PER_W // R
LANES = 16
GU = 8
GPR = D // (LANES * GU)
GPR_BITS = GPR.bit_length() - 1

_mesh = plsc.VectorSubcoreMesh(core_axis_name="c", subcore_axis_name="s")


@functools.partial(
    pl.kernel,
    mesh=_mesh,
    out_type=jax.ShapeDtypeStruct((B, S, D), jnp.float32),
    scratch_types=[
        pltpu.VMEM((R, D), jnp.float32),
        pltpu.VMEM((R, D), jnp.float32),
        pltpu.VMEM((R, D), jnp.float32),
        pltpu.VMEM((R, D), jnp.float32),
        pltpu.VMEM((R, D), jnp.float32),
        pltpu.VMEM((R, D), jnp.float32),
        pltpu.SemaphoreType.DMA,
        pltpu.SemaphoreType.DMA,
        pltpu.SemaphoreType.DMA,
        pltpu.SemaphoreType.DMA,
        pltpu.SemaphoreType.DMA,
        pltpu.SemaphoreType.DMA,
    ],
)
def _pe_add(x_hbm, pe_hbm, out_hbm, xb0, xb1, ob0, ob1, pb0, pb1,
            sx0, sx1, so0, so1, sp0, sp1):
    wid = lax.axis_index("s") * NC + lax.axis_index("c")
    row0 = wid * ROWS_PER_W

    xb, ob, pb = [xb0, xb1], [ob0, ob1], [pb0, pb1]
    sx, so, sp = [sx0, sx1], [so0, so1], [sp0, sp1]

    steps = [(c, b) for c in range(NCHUNK) for b in range(B)]
    T = len(steps)

    def xload(t):
        c, b = steps[t]
        return pltpu.make_async_copy(
            x_hbm.at[b, pl.ds(row0 + c * R, R), :], xb[t % 2], sx[t % 2])

    def peload(c):
        return pltpu.make_async_copy(
            pe_hbm.at[pl.ds(row0 + c * R, R), :], pb[c % 2], sp[c % 2])

    def store(t):
        c, b = steps[t]
        return pltpu.make_async_copy(
            ob[t % 2], out_hbm.at[b, pl.ds(row0 + c * R, R), :], so[t % 2])

    pending_x = {}
    pending_pe = {}
    pending_o = {}

    pending_pe[0] = peload(0)
    pending_pe[0].start()
    pending_x[0] = xload(0)
    pending_x[0].start()

    for t in range(T):
        c, b = steps[t]
        pending_x.pop(t).wait()
        if b == 0:
            pending_pe.pop(c).wait()
        if t + 1 < T:
            pending_x[t + 1] = xload(t + 1)
            pending_x[t + 1].start()
        if b == 0 and c + 1 < NCHUNK:
            pending_pe[c + 1] = peload(c + 1)
            pending_pe[c + 1].start()
        if t >= 2:
            pending_o.pop(t - 2).wait()

        xbuf, obuf, pbuf = xb[t % 2], ob[t % 2], pb[c % 2]

        def grp_body(i, carry, xbuf=xbuf, obuf=obuf, pbuf=pbuf):
            r = lax.shift_right_logical(i, GPR_BITS)
            gb = lax.bitwise_and(i, GPR - 1) * (LANES * GU)
            for u in range(GU):
                sl = pl.ds(gb + u * LANES, LANES)
                obuf[r, sl] = xbuf[r, sl] + pbuf[r, sl]
            return carry

        lax.fori_loop(0, R * GPR, grp_body, 0)

        pending_o[t] = store(t)
        pending_o[t].start()

    pending_o.pop(T - 2).wait()
    pending_o.pop(T - 1).wait()


def kernel(x, pe_table):
    return _pe_add(x, pe_table)
